# Initial kernel scaffold; baseline (speedup 1.0000x reference)
import jax
import jax.numpy as jnp
from jax.experimental import pallas as pl


def kernel(data_hypo, length_hypo, data_prem, length_prem, emb_hypo, emb_prem, W1, b1, W2, b2):
    raise NotImplementedError("write your pallas kernel here")


# speedup vs baseline: 2.9483x; 2.9483x over previous
import jax
import jax.numpy as jnp
from jax import lax
from jax.experimental import pallas as pl
from jax.experimental.pallas import tpu as pltpu
from jax.experimental.pallas import tpu_sc as plsc

B, L, V, D, H, O = 16384, 50, 1000000, 64, 128, 3

NC, NS = 2, 16
NW = NC * NS
BPW = B // NW
CE = 2
NI = CE * L
NCHUNK = BPW // CE
NBUF = 2


def _pool_body(idx_h_hbm, idx_p_hbm, emb_h_hbm, emb_p_hbm, out_hbm,
               idx_h_v, idx_p_v, rows_h, rows_p, xbuf,
               sem_h0, sem_h1, sem_p0, sem_p1):
    sems_h = (sem_h0, sem_h1)
    sems_p = (sem_p0, sem_p1)
    cid = lax.axis_index("c")
    sid = lax.axis_index("s")
    wid = sid * NC + cid

    pltpu.sync_copy(idx_h_hbm.at[wid], idx_h_v)
    pltpu.sync_copy(idx_p_hbm.at[wid], idx_p_v)

    def copies(g, b):
        ch = pltpu.make_async_copy(emb_h_hbm.at[idx_h_v.at[g]],
                                   rows_h.at[b], sems_h[b])
        cp = pltpu.make_async_copy(emb_p_hbm.at[idx_p_v.at[g]],
                                   rows_p.at[b], sems_p[b])
        return ch, cp

    def start(g, b):
        ch, cp = copies(g, b)
        ch.start()
        cp.start()

    def wait(g, b):
        ch, cp = copies(g, b)
        ch.wait()
        cp.wait()

    def accum(g, b):
        for j in range(CE):
            base = j * L

            def body(l2, acc):
                new = list(acc)
                for u in range(2):
                    l = base + l2 * 2 + u
                    for k in range(D // 16):
                        new[k] = (new[k]
                                  + rows_h[b, l, pl.ds(16 * k, 16)]
                                  + rows_p[b, l, pl.ds(16 * k, 16)])
                return tuple(new)

            zero = jnp.zeros((16,), jnp.float32)
            acc = lax.fori_loop(0, L // 2, body, (zero,) * (D // 16))
            row = g * CE + j
            for k in range(D // 16):
                xbuf[row, pl.ds(16 * k, 16)] = acc[k]

    for b in range(NBUF):
        start(b, b)

    def outer(i, carry):
        g0 = i * NBUF
        for b in range(NBUF):
            g = g0 + b
            wait(g, b)
            accum(g, b)
            start(g + NBUF, b)
        return carry

    lax.fori_loop(0, (NCHUNK - NBUF) // NBUF, outer, 0)

    for b in range(NBUF):
        g = NCHUNK - NBUF + b
        wait(g, b)
        accum(g, b)

    pltpu.sync_copy(xbuf, out_hbm.at[pl.ds(wid * BPW, BPW)])


_pool = pl.kernel(
    _pool_body,
    out_type=jax.ShapeDtypeStruct((B, D), jnp.float32),
    mesh=plsc.VectorSubcoreMesh(core_axis_name="c", subcore_axis_name="s"),
    compiler_params=pltpu.CompilerParams(use_tc_tiling_on_sc=False),
    scratch_types=[
        pltpu.VMEM((NCHUNK, NI), jnp.int32),
        pltpu.VMEM((NCHUNK, NI), jnp.int32),
        pltpu.VMEM((NBUF, NI, D), jnp.float32),
        pltpu.VMEM((NBUF, NI, D), jnp.float32),
        pltpu.VMEM((BPW, D), jnp.float32),
        pltpu.SemaphoreType.DMA,
        pltpu.SemaphoreType.DMA,
        pltpu.SemaphoreType.DMA,
        pltpu.SemaphoreType.DMA,
    ],
)


BLK = 1024


def _mlp_body(x_ref, w1_ref, b1_ref, w2_ref, b2_ref, o_ref):
    x = x_ref[...]
    h = lax.dot_general(x, w1_ref[...], (((1,), (0,)), ((), ())),
                        preferred_element_type=jnp.float32)
    h = jnp.maximum(h + b1_ref[...], 0.0)
    o = lax.dot_general(h, w2_ref[...], (((1,), (0,)), ((), ())),
                        preferred_element_type=jnp.float32)
    o_ref[...] = o + b2_ref[...]


_mlp = pl.pallas_call(
    _mlp_body,
    grid=(B // BLK,),
    in_specs=[
        pl.BlockSpec((BLK, D), lambda i: (i, 0)),
        pl.BlockSpec((D, H), lambda i: (0, 0)),
        pl.BlockSpec((1, H), lambda i: (0, 0)),
        pl.BlockSpec((H, 128), lambda i: (0, 0)),
        pl.BlockSpec((1, 128), lambda i: (0, 0)),
    ],
    out_specs=pl.BlockSpec((BLK, 128), lambda i: (i, 0)),
    out_shape=jax.ShapeDtypeStruct((B, 128), jnp.float32),
)


def kernel(data_hypo, length_hypo, data_prem, length_prem,
           emb_hypo, emb_prem, W1, b1, W2, b2):
    idx_h = data_hypo.reshape(NW, NCHUNK, NI)
    idx_p = data_prem.reshape(NW, NCHUNK, NI)
    pooled = _pool(idx_h, idx_p, emb_hypo, emb_prem)
    w1t = W1.T * (1.0 / L)
    w2p = jnp.zeros((H, 128), jnp.float32).at[:, :O].set(W2.T)
    b2p = jnp.zeros((1, 128), jnp.float32).at[0, :O].set(b2)
    out = _mlp(pooled, w1t, b1[None, :], w2p, b2p)
    return out[:, :O]
